# Initial kernel scaffold; baseline (speedup 1.0000x reference)
import jax
import jax.numpy as jnp
from jax.experimental import pallas as pl


def kernel(vertices, directions_l, directions_g0, weights_g1, bias_g1, directions_g1, gamma_l, beta_l, gamma_g0, beta_g0, gamma_g1, beta_g1):
    raise NotImplementedError("write your pallas kernel here")


# speedup vs baseline: 2.2521x; 2.2521x over previous
import jax
import jax.numpy as jnp
from jax import lax
from jax.experimental import pallas as pl

BS = 2
V = 1024
DIM = 128
K_L = 11
K_G = 101
IB = 128
JC = 128
NEG = -3e38


def _sortable(d):
    i = lax.bitcast_convert_type(d, jnp.int32)
    return jnp.where(i >= 0, i, i ^ jnp.int32(0x7FFFFFFF))


def _bfr(a):
    u = lax.bitcast_convert_type(a, jnp.uint32)
    u = (u + jnp.uint32(0x7FFF) + ((u >> 16) & jnp.uint32(1))) \
        & jnp.uint32(0xFFFF0000)
    return lax.bitcast_convert_type(u, jnp.float32)


def _mid(lo, hi):
    return (lo >> 1) + (hi >> 1) + (lo & hi & 1)


def _norm_dirs(dref):
    dirs = dref[...]
    n = jnp.sqrt(jnp.sum(dirs * dirs, axis=0, keepdims=True))
    return _bfr(dirs / jnp.maximum(n, 1e-12))


def _dist_select_kernel(vt_ref, vc_ref, keys_ref, tl_ref, tg_ref, am_ref):
    vt = vt_ref[0]
    x = vt[0:1, :]
    y = vt[1:2, :]
    z = vt[2:3, :]
    vc = vc_ref[0]
    xi = vc[:, 0:1]
    yi = vc[:, 1:2]
    zi = vc[:, 2:3]
    inner = _bfr(xi) * _bfr(x) + _bfr(yi) * _bfr(y) + _bfr(zi) * _bfr(z)
    quad_j = x * x + y * y + z * z
    quad_i = xi * xi + yi * yi + zi * zi
    d = (-2.0 * inner + quad_j) + quad_i
    keys = _sortable(d)
    keys_ref[0] = keys

    def body(_, carry):
        lo_l, hi_l, lo_g, hi_g = carry
        m_l = _mid(lo_l, hi_l)
        m_g = _mid(lo_g, hi_g)
        c_l = jnp.sum(jnp.where(keys <= m_l, 1.0, 0.0), axis=1, keepdims=True)
        c_g = jnp.sum(jnp.where(keys <= m_g, 1.0, 0.0), axis=1, keepdims=True)
        ok_l = c_l >= float(K_L)
        ok_g = c_g >= float(K_G)
        hi_l = jnp.where(ok_l, m_l, hi_l)
        lo_l = jnp.where(ok_l, lo_l, m_l + 1)
        hi_g = jnp.where(ok_g, m_g, hi_g)
        lo_g = jnp.where(ok_g, lo_g, m_g + 1)
        return lo_l, hi_l, lo_g, hi_g

    imin = jnp.full((V, 1), -2147483648, jnp.int32)
    imax = jnp.full((V, 1), 2147483647, jnp.int32)
    _, hi_l, _, hi_g = lax.fori_loop(0, 32, body, (imin, imax, imin, imax))
    tl_ref[0] = jnp.broadcast_to(hi_l, (V, DIM))
    tg_ref[0] = jnp.broadcast_to(hi_g, (V, DIM))
    rmin = jnp.min(keys, axis=1, keepdims=True)
    jidx = lax.broadcasted_iota(jnp.int32, (V, V), 1)
    amin = jnp.min(jnp.where(keys == rmin, jidx, V), axis=1, keepdims=True)
    am_ref[0] = jnp.broadcast_to(amin, (V, DIM))


def _ndn_chunk(vt_ref, vci, j0):
    xi = vci[:, 0:1]
    yi = vci[:, 1:2]
    zi = vci[:, 2:3]
    dx = vt_ref[0, 0:1, pl.ds(j0, JC)] - xi
    dy = vt_ref[0, 1:2, pl.ds(j0, JC)] - yi
    dz = vt_ref[0, 2:3, pl.ds(j0, JC)] - zi
    n = jnp.sqrt(dx * dx + dy * dy + dz * dz)
    n = jnp.maximum(n, 1e-12)
    return _bfr(dx / n), _bfr(dy / n), _bfr(dz / n)


def _surface_kernel(vt_ref, vc_ref, keys_ref, tl_ref, tg_ref, am_ref,
                    dl_ref, dg0_ref, ol_ref, og_ref):
    vci = vc_ref[0]
    tl = tl_ref[0][:, 0:1]
    tg = tg_ref[0][:, 0:1]
    am = am_ref[0][:, 0:1]
    sl = _norm_dirs(dl_ref)
    sg = _norm_dirs(dg0_ref)
    slx, sly, slz = sl[0:1, :], sl[1:2, :], sl[2:3, :]
    sgx, sgy, sgz = sg[0:1, :], sg[1:2, :], sg[2:3, :]

    def chunk(c, carry):
        acc_l, acc_g = carry
        j0 = c * JC
        nx, ny, nz = _ndn_chunk(vt_ref, vci, j0)
        kc = keys_ref[0, :, pl.ds(j0, JC)]
        jidx = j0 + lax.broadcasted_iota(jnp.int32, (IB, JC), 1)
        keep = jidx != am
        m_l = jnp.where((kc <= tl) & keep, 1.0, 0.0)
        m_g = jnp.where((kc <= tg) & keep, 1.0, 0.0)
        nx3 = nx[:, :, None]
        ny3 = ny[:, :, None]
        nz3 = nz[:, :, None]
        th_l = jnp.maximum(
            nx3 * slx[None] + ny3 * sly[None] + nz3 * slz[None], 0.0)
        acc_l = jnp.maximum(acc_l, jnp.max(th_l * m_l[:, :, None], axis=1))
        th_g = jnp.maximum(
            nx3 * sgx[None] + ny3 * sgy[None] + nz3 * sgz[None], 0.0)
        acc_g = jnp.maximum(acc_g, jnp.max(th_g * m_g[:, :, None], axis=1))
        return acc_l, acc_g

    z0 = jnp.zeros((IB, DIM), jnp.float32)
    acc_l, acc_g = lax.fori_loop(0, V // JC, chunk, (z0, z0))
    ol_ref[0] = acc_l
    og_ref[0] = acc_g


def _bn1_kernel(ol_ref, og_ref, w_ref, b_ref, gl_ref, bl_ref, gg_ref, bg_ref,
                fml_ref, ctr_ref, fs_ref):
    def _bn_relu(x, gamma, beta):
        m = jnp.mean(x, axis=0, keepdims=True)
        v = jnp.mean((x - m) * (x - m), axis=0, keepdims=True)
        return jnp.maximum(gamma * (x - m) / jnp.sqrt(v + 1e-5) + beta, 0.0)

    xl = ol_ref[...].reshape(BS * V, DIM)
    xg = og_ref[...].reshape(BS * V, DIM)
    fml = _bn_relu(xl, gl_ref[...], bl_ref[...])
    fmg = _bn_relu(xg, gg_ref[...], bg_ref[...])
    fml_ref[...] = fml.reshape(BS, V, DIM)
    fo = jnp.dot(_bfr(fmg), _bfr(w_ref[...]),
                 preferred_element_type=jnp.float32) + b_ref[...]
    ctr_ref[...] = fo[:, :DIM].reshape(BS, V, DIM)
    fs_ref[...] = fo[:, DIM:].reshape(BS, V, DIM)


def _layer_kernel(vt_ref, vc_ref, keys_ref, tg_ref, am_ref, dg1_ref, fs_ref,
                  ctr_ref, o2_ref):
    vci = vc_ref[0]
    tg = tg_ref[0][:, 0:1]
    am = am_ref[0][:, 0:1]
    sg = _norm_dirs(dg1_ref)
    sgx, sgy, sgz = sg[0:1, :], sg[1:2, :], sg[2:3, :]

    def chunk(c, acc):
        j0 = c * JC
        nx, ny, nz = _ndn_chunk(vt_ref, vci, j0)
        kc = keys_ref[0, :, pl.ds(j0, JC)]
        jidx = j0 + lax.broadcasted_iota(jnp.int32, (IB, JC), 1)
        mask = (kc <= tg) & (jidx != am)
        pen = jnp.where(mask, 0.0, NEG)
        theta = jnp.maximum(
            nx[:, :, None] * sgx[None] + ny[:, :, None] * sgy[None]
            + nz[:, :, None] * sgz[None], 0.0)
        fsj = fs_ref[0, pl.ds(j0, JC), :]
        act = theta * fsj[None, :, :] + pen[:, :, None]
        return jnp.maximum(acc, jnp.max(act, axis=1))

    acc = lax.fori_loop(0, V // JC, chunk, jnp.full((IB, DIM), NEG, jnp.float32))
    o2_ref[0] = ctr_ref[0] + acc


def _bn2_kernel(o2_ref, fml_ref, gg_ref, bg_ref, out_ref):
    x = o2_ref[...].reshape(BS * V, DIM)
    m = jnp.mean(x, axis=0, keepdims=True)
    v = jnp.mean((x - m) * (x - m), axis=0, keepdims=True)
    fmg = jnp.maximum(gg_ref[...] * (x - m) / jnp.sqrt(v + 1e-5) + bg_ref[...], 0.0)
    out_ref[:, :, :DIM] = fml_ref[...]
    out_ref[:, :, DIM:] = fmg.reshape(BS, V, DIM)


def kernel(vertices, directions_l, directions_g0, weights_g1, bias_g1,
           directions_g1, gamma_l, beta_l, gamma_g0, beta_g0, gamma_g1,
           beta_g1):
    f32 = jnp.float32
    vt = jnp.transpose(vertices, (0, 2, 1))
    vc = vertices

    keys, t_l, t_g, amin = pl.pallas_call(
        _dist_select_kernel,
        grid=(BS,),
        in_specs=[
            pl.BlockSpec((1, 3, V), lambda b: (b, 0, 0)),
            pl.BlockSpec((1, V, 3), lambda b: (b, 0, 0)),
        ],
        out_specs=[
            pl.BlockSpec((1, V, V), lambda b: (b, 0, 0)),
            pl.BlockSpec((1, V, DIM), lambda b: (b, 0, 0)),
            pl.BlockSpec((1, V, DIM), lambda b: (b, 0, 0)),
            pl.BlockSpec((1, V, DIM), lambda b: (b, 0, 0)),
        ],
        out_shape=[
            jax.ShapeDtypeStruct((BS, V, V), jnp.int32),
            jax.ShapeDtypeStruct((BS, V, DIM), jnp.int32),
            jax.ShapeDtypeStruct((BS, V, DIM), jnp.int32),
            jax.ShapeDtypeStruct((BS, V, DIM), jnp.int32),
        ],
    )(vt, vc)

    nib = V // IB
    o_l, o_g = pl.pallas_call(
        _surface_kernel,
        grid=(BS, nib),
        in_specs=[
            pl.BlockSpec((1, 3, V), lambda b, i: (b, 0, 0)),
            pl.BlockSpec((1, IB, 3), lambda b, i: (b, i, 0)),
            pl.BlockSpec((1, IB, V), lambda b, i: (b, i, 0)),
            pl.BlockSpec((1, IB, DIM), lambda b, i: (b, i, 0)),
            pl.BlockSpec((1, IB, DIM), lambda b, i: (b, i, 0)),
            pl.BlockSpec((1, IB, DIM), lambda b, i: (b, i, 0)),
            pl.BlockSpec((3, DIM), lambda b, i: (0, 0)),
            pl.BlockSpec((3, DIM), lambda b, i: (0, 0)),
        ],
        out_specs=[
            pl.BlockSpec((1, IB, DIM), lambda b, i: (b, i, 0)),
            pl.BlockSpec((1, IB, DIM), lambda b, i: (b, i, 0)),
        ],
        out_shape=[
            jax.ShapeDtypeStruct((BS, V, DIM), f32),
            jax.ShapeDtypeStruct((BS, V, DIM), f32),
        ],
    )(vt, vc, keys, t_l, t_g, amin, directions_l, directions_g0)

    fm_l, center, fsup = pl.pallas_call(
        _bn1_kernel,
        in_specs=[
            pl.BlockSpec((BS, V, DIM), lambda: (0, 0, 0)),
            pl.BlockSpec((BS, V, DIM), lambda: (0, 0, 0)),
            pl.BlockSpec((DIM, 2 * DIM), lambda: (0, 0)),
            pl.BlockSpec((1, 2 * DIM), lambda: (0, 0)),
            pl.BlockSpec((1, DIM), lambda: (0, 0)),
            pl.BlockSpec((1, DIM), lambda: (0, 0)),
            pl.BlockSpec((1, DIM), lambda: (0, 0)),
            pl.BlockSpec((1, DIM), lambda: (0, 0)),
        ],
        out_specs=[
            pl.BlockSpec((BS, V, DIM), lambda: (0, 0, 0)),
            pl.BlockSpec((BS, V, DIM), lambda: (0, 0, 0)),
            pl.BlockSpec((BS, V, DIM), lambda: (0, 0, 0)),
        ],
        out_shape=[
            jax.ShapeDtypeStruct((BS, V, DIM), f32),
            jax.ShapeDtypeStruct((BS, V, DIM), f32),
            jax.ShapeDtypeStruct((BS, V, DIM), f32),
        ],
    )(o_l, o_g, weights_g1, bias_g1.reshape(1, 2 * DIM),
      gamma_l.reshape(1, DIM), beta_l.reshape(1, DIM),
      gamma_g0.reshape(1, DIM), beta_g0.reshape(1, DIM))

    o2 = pl.pallas_call(
        _layer_kernel,
        grid=(BS, nib),
        in_specs=[
            pl.BlockSpec((1, 3, V), lambda b, i: (b, 0, 0)),
            pl.BlockSpec((1, IB, 3), lambda b, i: (b, i, 0)),
            pl.BlockSpec((1, IB, V), lambda b, i: (b, i, 0)),
            pl.BlockSpec((1, IB, DIM), lambda b, i: (b, i, 0)),
            pl.BlockSpec((1, IB, DIM), lambda b, i: (b, i, 0)),
            pl.BlockSpec((3, DIM), lambda b, i: (0, 0)),
            pl.BlockSpec((1, V, DIM), lambda b, i: (b, 0, 0)),
            pl.BlockSpec((1, IB, DIM), lambda b, i: (b, i, 0)),
        ],
        out_specs=pl.BlockSpec((1, IB, DIM), lambda b, i: (b, i, 0)),
        out_shape=jax.ShapeDtypeStruct((BS, V, DIM), f32),
    )(vt, vc, keys, t_g, amin, directions_g1, fsup, center)

    out = pl.pallas_call(
        _bn2_kernel,
        in_specs=[
            pl.BlockSpec((BS, V, DIM), lambda: (0, 0, 0)),
            pl.BlockSpec((BS, V, DIM), lambda: (0, 0, 0)),
            pl.BlockSpec((1, DIM), lambda: (0, 0)),
            pl.BlockSpec((1, DIM), lambda: (0, 0)),
        ],
        out_specs=pl.BlockSpec((BS, V, 2 * DIM), lambda: (0, 0, 0)),
        out_shape=jax.ShapeDtypeStruct((BS, V, 2 * DIM), f32),
    )(o2, fm_l, gamma_g1.reshape(1, DIM), beta_g1.reshape(1, DIM))
    return out
